# baseline (speedup 1.0000x reference)
import functools

import jax
import jax.numpy as jnp
from jax import lax
from jax.experimental import pallas as pl
from jax.experimental.pallas import tpu as pltpu
from jax.experimental.pallas import tpu_sc as plsc

DIM = 64
LANES = 16
NUM_CORES = 2
NUM_SUBCORES = 16
NUM_WORKERS = NUM_CORES * NUM_SUBCORES
CHUNK = 256
SUB = CHUNK // 128
NBUF = 3
GROUPS = 8
BLOCK = GROUPS * LANES
EPS = 1e-12


def _body(ids2_hbm, table_hbm, gamma_hbm, beta_hbm, out_hbm,
          idx_v, rows_v, obuf_v, gamma_v, beta_v, gsem, osem,
          *, per_w, n_chunks):
    wid = lax.axis_index("s") * NUM_CORES + lax.axis_index("c")
    base = wid * per_w
    base_c = wid * (per_w // 128)

    pltpu.sync_copy(gamma_hbm, gamma_v)
    pltpu.sync_copy(beta_hbm, beta_v)
    gs = [gamma_v[pl.ds(j * LANES, LANES)] for j in range(DIM // LANES)]
    bs = [beta_v[pl.ds(j * LANES, LANES)] for j in range(DIM // LANES)]
    lane = lax.iota(jnp.int32, LANES)

    def fire_gather(c, buf):
        pltpu.sync_copy(ids2_hbm.at[pl.ds(base_c + c * SUB, SUB)],
                        idx_v.at[buf])
        for j in range(SUB):
            pltpu.async_copy(
                table_hbm.at[idx_v.at[buf, j]],
                rows_v.at[buf, pl.ds(j * 128, 128)],
                gsem.at[buf],
            )

    def drain_gather(buf):
        for j in range(SUB):
            pltpu.make_async_copy(
                table_hbm.at[idx_v.at[buf, j]],
                rows_v.at[buf, pl.ds(j * 128, 128)],
                gsem.at[buf],
            ).wait()

    def fire_out(c, ob):
        pltpu.async_copy(
            obuf_v.at[ob],
            out_hbm.at[pl.ds(base + c * CHUNK, CHUNK), pl.ds(0, DIM)],
            osem.at[ob],
        )

    def wait_out(c, ob):
        pltpu.make_async_copy(
            obuf_v.at[ob],
            out_hbm.at[pl.ds(base + c * CHUNK, CHUNK), pl.ds(0, DIM)],
            osem.at[ob],
        ).wait()

    def ln_rows(buf, ob):
        bufv = jnp.full((LANES,), buf, jnp.int32)
        obv = jnp.full((LANES,), ob, jnp.int32)

        def block_body(blk, carry):
            rowvs = [blk * BLOCK + g * LANES + lane for g in range(GROUPS)]

            def p1_body(j, sums):
                jv = lax.bitwise_and(lane + j, DIM - 1)
                out = []
                for g in range(GROUPS):
                    v = plsc.load_gather(rows_v, [bufv, rowvs[g], jv])
                    out.append(sums[g] + v)
                    out.append(sums[GROUPS + g] + v * v)
                return tuple(out[0::2]) + tuple(out[1::2])

            zero = jnp.zeros((LANES,), jnp.float32)
            sums = lax.fori_loop(0, DIM, p1_body, (zero,) * (2 * GROUPS))

            scales = []
            shifts = []
            for g in range(GROUPS):
                mv = sums[g] * (1.0 / DIM)
                varv = jnp.maximum(
                    sums[GROUPS + g] * (1.0 / DIM) - mv * mv, 0.0) + EPS
                bits = 0x5F3759DF - lax.shift_right_logical(
                    plsc.bitcast(varv, jnp.int32), 1)
                y = plsc.bitcast(bits, jnp.float32)
                y = y * (1.5 - (0.5 * varv) * (y * y))
                y = y * (1.5 - (0.5 * varv) * (y * y))
                scales.append(y)
                shifts.append(-mv * y)

            def p2_body(j2, carry2):
                for u in range(2):
                    j = j2 * 2 + u
                    jv = lax.bitwise_and(lane + j, DIM - 1)
                    gj = plsc.load_gather(gamma_v, [jv])
                    bj = plsc.load_gather(beta_v, [jv])
                    vals = [plsc.load_gather(rows_v, [bufv, rowvs[g], jv])
                            for g in range(GROUPS)]
                    outs = [(vals[g] * scales[g] + shifts[g]) * gj + bj
                            for g in range(GROUPS)]
                    for g in range(GROUPS):
                        plsc.store_scatter(obuf_v, [obv, rowvs[g], jv],
                                           outs[g])
                return carry2

            lax.fori_loop(0, DIM // 2, p2_body, 0)
            return carry

        lax.fori_loop(0, CHUNK // BLOCK, block_body, 0)

    fire_gather(0, 0)
    fire_gather(1, 1)

    def chunk_body(c, carry):
        buf = lax.rem(c, NBUF)
        buf2 = lax.rem(c + 2, NBUF)
        ob = lax.rem(c, 2)
        drain_gather(buf)

        @pl.when(c >= 2)
        def _():
            wait_out(c - 2, ob)

        ln_rows(buf, ob)
        fire_out(c, ob)

        @pl.when(c + 2 < n_chunks)
        def _():
            fire_gather(c + 2, buf2)

        return carry

    lax.fori_loop(0, n_chunks, chunk_body, 0)
    wait_out(n_chunks - 2, lax.rem(n_chunks - 2, 2))
    wait_out(n_chunks - 1, lax.rem(n_chunks - 1, 2))


def _launch(ids2, table2, gamma, beta):
    n = ids2.shape[0] * ids2.shape[1]
    per_w = n // NUM_WORKERS
    n_chunks = per_w // CHUNK
    mesh = plsc.VectorSubcoreMesh(core_axis_name="c", subcore_axis_name="s")
    kfn = pl.kernel(
        functools.partial(_body, per_w=per_w, n_chunks=n_chunks),
        out_type=jax.ShapeDtypeStruct((n, 128), jnp.float32),
        mesh=mesh,
        scratch_types=[
            pltpu.VMEM((NBUF, SUB, 128), jnp.int32),
            pltpu.VMEM((NBUF, CHUNK, DIM), jnp.float32),
            pltpu.VMEM((2, CHUNK, DIM), jnp.float32),
            pltpu.VMEM((DIM,), jnp.float32),
            pltpu.VMEM((DIM,), jnp.float32),
            pltpu.SemaphoreType.DMA((NBUF,)),
            pltpu.SemaphoreType.DMA((2,)),
        ],
        compiler_params=pltpu.CompilerParams(
            needs_layout_passes=False, use_tc_tiling_on_sc=False
        ),
    )
    return kfn(ids2, table2, gamma, beta)


def _transpose_body(tt_hbm, tail_hbm, out_hbm, in_v, tr_v, isem, osem,
                    *, nvt, tail_vt):
    wid = lax.axis_index("s") * NUM_CORES + lax.axis_index("c")
    lane = lax.iota(jnp.int32, LANES)
    rowts = [lax.shift_right_logical(k * LANES + lane, 1) for k in range(8)]
    colbs = [lax.shift_left(lax.bitwise_and(k * LANES + lane, 1), 6)
             for k in range(8)]
    vis = [k * LANES + lane for k in range(8)]
    n_mine = (nvt - wid + NUM_WORKERS - 1) // NUM_WORKERS

    def fire_in(i, buf):
        vt = wid + i * NUM_WORKERS
        pltpu.async_copy(tt_hbm.at[:, pl.ds(vt * 128, 128)], in_v.at[buf],
                        isem.at[buf])

    def vt_body(i, carry):
        buf = lax.rem(i, 2)
        vt = wid + i * NUM_WORKERS
        pltpu.make_async_copy(
            tt_hbm.at[:, pl.ds(vt * 128, 128)], in_v.at[buf],
            isem.at[buf]).wait()

        @pl.when(i >= 2)
        def _():
            pltpu.make_async_copy(
                tr_v.at[buf], out_hbm.at[pl.ds((vt - 2 * NUM_WORKERS) * 64, 64)],
                osem.at[buf]).wait()

        bufv = jnp.full((LANES,), buf, jnp.int32)

        def s_body(s, carry2):
            cvec = lax.bitwise_and(lane + s, DIM - 1)
            vals = [plsc.load_gather(in_v, [bufv, cvec, vis[k]])
                    for k in range(8)]
            for k in range(8):
                plsc.store_scatter(tr_v, [bufv, rowts[k], colbs[k] + cvec],
                                   vals[k])
            return carry2

        lax.fori_loop(0, DIM, s_body, 0)
        pltpu.async_copy(tr_v.at[buf], out_hbm.at[pl.ds(vt * 64, 64)],
                        osem.at[buf])

        @pl.when(i + 2 < n_mine)
        def _():
            fire_in(i + 2, buf)

        return carry

    @pl.when(n_mine >= 1)
    def _():
        fire_in(0, 0)

    @pl.when(n_mine >= 2)
    def _():
        fire_in(1, 1)

    lax.fori_loop(0, n_mine, vt_body, 0)

    def drain_tail(i, carry):
        vt = wid + i * NUM_WORKERS
        pltpu.make_async_copy(
            tr_v.at[lax.rem(i, 2)], out_hbm.at[pl.ds(vt * 64, 64)],
            osem.at[lax.rem(i, 2)]).wait()
        return carry

    lax.fori_loop(lax.max(n_mine - 2, 0), n_mine, drain_tail, 0)

    @pl.when(wid == 0)
    def _():
        pltpu.sync_copy(tail_hbm, in_v.at[0, pl.ds(0, 32)])
        pltpu.sync_copy(in_v.at[0, pl.ds(0, 32)],
                        out_hbm.at[pl.ds(tail_vt * 64, 32)])


def _relayout_table(table):
    v = table.shape[0]
    nvt = v // 128
    tail = table[nvt * 128:].reshape(32, 128)
    mesh = plsc.VectorSubcoreMesh(core_axis_name="c", subcore_axis_name="s")
    kfn = pl.kernel(
        functools.partial(_transpose_body, nvt=nvt, tail_vt=nvt),
        out_type=jax.ShapeDtypeStruct((v // 2, 128), jnp.float32),
        mesh=mesh,
        scratch_types=[
            pltpu.VMEM((2, DIM, 128), jnp.float32),
            pltpu.VMEM((2, DIM, 128), jnp.float32),
            pltpu.SemaphoreType.DMA((2,)),
            pltpu.SemaphoreType.DMA((2,)),
        ],
        compiler_params=pltpu.CompilerParams(
            needs_layout_passes=False, use_tc_tiling_on_sc=True
        ),
    )
    return kfn(table.T, tail)


def kernel(token_ids, table, gamma, beta):
    b, s = token_ids.shape
    n = b * s
    v = table.shape[0]
    ids2 = token_ids.astype(jnp.int32).reshape(n // 128, 128)
    table_p = _relayout_table(table).reshape(v, DIM)
    outp = _launch(ids2, table_p, gamma, beta)
    return outp.reshape(b, s, 128)[:, :, :DIM]


# speedup vs baseline: 1.0901x; 1.0901x over previous
import functools

import jax
import jax.numpy as jnp
from jax import lax
from jax.experimental import pallas as pl
from jax.experimental.pallas import tpu as pltpu
from jax.experimental.pallas import tpu_sc as plsc

DIM = 64
LANES = 16
NUM_CORES = 2
NUM_SUBCORES = 16
NUM_WORKERS = NUM_CORES * NUM_SUBCORES
CHUNK = 256
SUB = CHUNK // 128
NBUF = 3
GROUPS = 8
BLOCK = GROUPS * LANES
EPS = 1e-12


def _body(ids2_hbm, table_hbm, gamma_hbm, beta_hbm, out_hbm,
          idx_v, rows_v, obuf_v, gamma_v, beta_v, gsem, osem,
          *, per_w, n_chunks):
    wid = lax.axis_index("s") * NUM_CORES + lax.axis_index("c")
    base = wid * per_w
    base_c = wid * (per_w // 128)

    pltpu.sync_copy(gamma_hbm, gamma_v)
    pltpu.sync_copy(beta_hbm, beta_v)
    gs = [gamma_v[pl.ds(j * LANES, LANES)] for j in range(DIM // LANES)]
    bs = [beta_v[pl.ds(j * LANES, LANES)] for j in range(DIM // LANES)]
    lane = lax.iota(jnp.int32, LANES)

    def fire_gather(c, buf):
        pltpu.sync_copy(ids2_hbm.at[pl.ds(base_c + c * SUB, SUB)],
                        idx_v.at[buf])
        for j in range(SUB):
            pltpu.async_copy(
                table_hbm.at[idx_v.at[buf, j]],
                rows_v.at[buf, pl.ds(j * 128, 128)],
                gsem.at[buf],
            )

    def drain_gather(buf):
        for j in range(SUB):
            pltpu.make_async_copy(
                table_hbm.at[idx_v.at[buf, j]],
                rows_v.at[buf, pl.ds(j * 128, 128)],
                gsem.at[buf],
            ).wait()

    def fire_out(c, ob):
        pltpu.async_copy(
            obuf_v.at[ob],
            out_hbm.at[pl.ds(base + c * CHUNK, CHUNK), pl.ds(0, DIM)],
            osem.at[ob],
        )

    def wait_out(c, ob):
        pltpu.make_async_copy(
            obuf_v.at[ob],
            out_hbm.at[pl.ds(base + c * CHUNK, CHUNK), pl.ds(0, DIM)],
            osem.at[ob],
        ).wait()

    def ln_rows(buf, ob):
        bufv = jnp.full((LANES,), buf, jnp.int32)
        obv = jnp.full((LANES,), ob, jnp.int32)

        def block_body(blk, carry):
            rowvs = [blk * BLOCK + g * LANES + lane for g in range(GROUPS)]

            def p1_body(j, sums):
                jv = lax.bitwise_and(lane + j, DIM - 1)
                out = []
                for g in range(GROUPS):
                    v = plsc.load_gather(rows_v, [bufv, rowvs[g], jv])
                    out.append(sums[g] + v)
                    out.append(sums[GROUPS + g] + v * v)
                return tuple(out[0::2]) + tuple(out[1::2])

            zero = jnp.zeros((LANES,), jnp.float32)
            sums = lax.fori_loop(0, DIM, p1_body, (zero,) * (2 * GROUPS))

            scales = []
            shifts = []
            for g in range(GROUPS):
                mv = sums[g] * (1.0 / DIM)
                varv = jnp.maximum(
                    sums[GROUPS + g] * (1.0 / DIM) - mv * mv, 0.0) + EPS
                bits = 0x5F3759DF - lax.shift_right_logical(
                    plsc.bitcast(varv, jnp.int32), 1)
                y = plsc.bitcast(bits, jnp.float32)
                y = y * (1.5 - (0.5 * varv) * (y * y))
                y = y * (1.5 - (0.5 * varv) * (y * y))
                scales.append(y)
                shifts.append(-mv * y)

            def p2_body(j, carry2):
                jv = lax.bitwise_and(lane + j, DIM - 1)
                gj = plsc.load_gather(gamma_v, [jv])
                bj = plsc.load_gather(beta_v, [jv])
                vals = [plsc.load_gather(rows_v, [bufv, rowvs[g], jv])
                        for g in range(GROUPS)]
                outs = [(vals[g] * scales[g] + shifts[g]) * gj + bj
                        for g in range(GROUPS)]
                for g in range(GROUPS):
                    plsc.store_scatter(obuf_v, [obv, rowvs[g], jv], outs[g])
                return carry2

            lax.fori_loop(0, DIM, p2_body, 0)
            return carry

        lax.fori_loop(0, CHUNK // BLOCK, block_body, 0)

    fire_gather(0, 0)
    fire_gather(1, 1)

    def chunk_body(c, carry):
        buf = lax.rem(c, NBUF)
        buf2 = lax.rem(c + 2, NBUF)
        ob = lax.rem(c, 2)
        drain_gather(buf)

        @pl.when(c >= 2)
        def _():
            wait_out(c - 2, ob)

        ln_rows(buf, ob)
        fire_out(c, ob)

        @pl.when(c + 2 < n_chunks)
        def _():
            fire_gather(c + 2, buf2)

        return carry

    lax.fori_loop(0, n_chunks, chunk_body, 0)
    wait_out(n_chunks - 2, lax.rem(n_chunks - 2, 2))
    wait_out(n_chunks - 1, lax.rem(n_chunks - 1, 2))


def _launch(ids2, table2, gamma, beta):
    n = ids2.shape[0] * ids2.shape[1]
    per_w = n // NUM_WORKERS
    n_chunks = per_w // CHUNK
    mesh = plsc.VectorSubcoreMesh(core_axis_name="c", subcore_axis_name="s")
    kfn = pl.kernel(
        functools.partial(_body, per_w=per_w, n_chunks=n_chunks),
        out_type=jax.ShapeDtypeStruct((n, 128), jnp.float32),
        mesh=mesh,
        scratch_types=[
            pltpu.VMEM((NBUF, SUB, 128), jnp.int32),
            pltpu.VMEM((NBUF, CHUNK, DIM), jnp.float32),
            pltpu.VMEM((2, CHUNK, DIM), jnp.float32),
            pltpu.VMEM((DIM,), jnp.float32),
            pltpu.VMEM((DIM,), jnp.float32),
            pltpu.SemaphoreType.DMA((NBUF,)),
            pltpu.SemaphoreType.DMA((2,)),
        ],
        compiler_params=pltpu.CompilerParams(
            needs_layout_passes=False, use_tc_tiling_on_sc=False
        ),
    )
    return kfn(ids2, table2, gamma, beta)


def _transpose_body(tt_hbm, tail_hbm, out_hbm, in_v, tr_v, isem, osem,
                    *, nvt, tail_vt):
    wid = lax.axis_index("s") * NUM_CORES + lax.axis_index("c")
    lane = lax.iota(jnp.int32, LANES)
    rowts = [lax.shift_right_logical(k * LANES + lane, 1) for k in range(8)]
    colbs = [lax.shift_left(lax.bitwise_and(k * LANES + lane, 1), 6)
             for k in range(8)]
    vis = [k * LANES + lane for k in range(8)]
    n_mine = (nvt - wid + NUM_WORKERS - 1) // NUM_WORKERS

    def fire_in(i, buf):
        vt = wid + i * NUM_WORKERS
        pltpu.async_copy(tt_hbm.at[:, pl.ds(vt * 128, 128)], in_v.at[buf],
                        isem.at[buf])

    def vt_body(i, carry):
        buf = lax.rem(i, 2)
        vt = wid + i * NUM_WORKERS
        pltpu.make_async_copy(
            tt_hbm.at[:, pl.ds(vt * 128, 128)], in_v.at[buf],
            isem.at[buf]).wait()

        @pl.when(i >= 2)
        def _():
            pltpu.make_async_copy(
                tr_v.at[buf], out_hbm.at[pl.ds((vt - 2 * NUM_WORKERS) * 64, 64)],
                osem.at[buf]).wait()

        bufv = jnp.full((LANES,), buf, jnp.int32)

        def s_body(s, carry2):
            cvec = lax.bitwise_and(lane + s, DIM - 1)
            vals = [plsc.load_gather(in_v, [bufv, cvec, vis[k]])
                    for k in range(8)]
            for k in range(8):
                plsc.store_scatter(tr_v, [bufv, rowts[k], colbs[k] + cvec],
                                   vals[k])
            return carry2

        lax.fori_loop(0, DIM, s_body, 0)
        pltpu.async_copy(tr_v.at[buf], out_hbm.at[pl.ds(vt * 64, 64)],
                        osem.at[buf])

        @pl.when(i + 2 < n_mine)
        def _():
            fire_in(i + 2, buf)

        return carry

    @pl.when(n_mine >= 1)
    def _():
        fire_in(0, 0)

    @pl.when(n_mine >= 2)
    def _():
        fire_in(1, 1)

    lax.fori_loop(0, n_mine, vt_body, 0)

    def drain_tail(i, carry):
        vt = wid + i * NUM_WORKERS
        pltpu.make_async_copy(
            tr_v.at[lax.rem(i, 2)], out_hbm.at[pl.ds(vt * 64, 64)],
            osem.at[lax.rem(i, 2)]).wait()
        return carry

    lax.fori_loop(lax.max(n_mine - 2, 0), n_mine, drain_tail, 0)

    @pl.when(wid == 0)
    def _():
        pltpu.sync_copy(tail_hbm, in_v.at[0, pl.ds(0, 32)])
        pltpu.sync_copy(in_v.at[0, pl.ds(0, 32)],
                        out_hbm.at[pl.ds(tail_vt * 64, 32)])


def _relayout_table(table):
    v = table.shape[0]
    nvt = v // 128
    tail = table[nvt * 128:].reshape(32, 128)
    mesh = plsc.VectorSubcoreMesh(core_axis_name="c", subcore_axis_name="s")
    kfn = pl.kernel(
        functools.partial(_transpose_body, nvt=nvt, tail_vt=nvt),
        out_type=jax.ShapeDtypeStruct((v // 2, 128), jnp.float32),
        mesh=mesh,
        scratch_types=[
            pltpu.VMEM((2, DIM, 128), jnp.float32),
            pltpu.VMEM((2, DIM, 128), jnp.float32),
            pltpu.SemaphoreType.DMA((2,)),
            pltpu.SemaphoreType.DMA((2,)),
        ],
        compiler_params=pltpu.CompilerParams(
            needs_layout_passes=False, use_tc_tiling_on_sc=True
        ),
    )
    return kfn(table.T, tail)


def kernel(token_ids, table, gamma, beta):
    b, s = token_ids.shape
    n = b * s
    v = table.shape[0]
    ids2 = token_ids.astype(jnp.int32).reshape(n // 128, 128)
    table_p = _relayout_table(table).reshape(v, DIM)
    outp = _launch(ids2, table_p, gamma, beta)
    return outp.reshape(b, s, 128)[:, :, :DIM]
